# baseline (speedup 1.0000x reference)
import functools

import jax
import jax.numpy as jnp
from jax import lax
from jax.experimental import pallas as pl
from jax.experimental.pallas import tpu as pltpu
from jax.experimental.pallas import tpu_sc as plsc

N = 50000
E = 800000
NT = 100
D = 64
NP = 51200
EP = 819200
PAD_NODE = 50001
NC = 2
NS = 16
SHARD = 12800
ASIZE = SHARD * NT
ATRASH = ASIZE
ZCH = 3200
EW_SCALE = -1.0 / 64.0


def _sc_degrees(srcp, dstp, ones_h, zeros_h):
    mesh = plsc.VectorSubcoreMesh(core_axis_name="c", subcore_axis_name="s")

    @functools.partial(
        pl.kernel,
        out_type=[
            jax.ShapeDtypeStruct((NP,), jnp.float32),
            jax.ShapeDtypeStruct((NP,), jnp.float32),
        ],
        mesh=mesh,
        scratch_types=[
            pltpu.VMEM((ZCH,), jnp.int32),
            pltpu.VMEM((ZCH,), jnp.float32),
            pltpu.VMEM((ZCH,), jnp.float32),
            pltpu.VMEM_SHARED((NP,), jnp.float32),
        ],
        compiler_params=pltpu.CompilerParams(needs_layout_passes=False),
    )
    def deg_kernel(src_r, dst_r, ones_r, zeros_r, dego_r, degi_r,
                   ebuf, ones_v, zeros_v, hist):
        c = lax.axis_index("c")
        s = lax.axis_index("s")
        pltpu.sync_copy(ones_r, ones_v)
        pltpu.sync_copy(zeros_r, zeros_v)
        pltpu.sync_copy(zeros_v, hist.at[pl.ds(s * ZCH, ZCH)])
        plsc.subcore_barrier()
        base = s * (EP // NS)
        for k in range(16):
            @pl.when(c == 0)
            def _():
                pltpu.sync_copy(src_r.at[pl.ds(base + k * ZCH, ZCH)], ebuf)

            @pl.when(c == 1)
            def _():
                pltpu.sync_copy(dst_r.at[pl.ds(base + k * ZCH, ZCH)], ebuf)

            pltpu.sync_copy(ones_v, hist.at[ebuf], add=True)
        plsc.subcore_barrier()

        @pl.when((c == 0) & (s == 0))
        def _():
            pltpu.sync_copy(hist, dego_r)

        @pl.when((c == 1) & (s == 0))
        def _():
            pltpu.sync_copy(hist, degi_r)

    return deg_kernel(srcp, dstp, ones_h, zeros_h)


def _tc_norms_table(dego2, degi2, af_table, W_embed, b_embed2, W1):
    def body(dego_r, degi_r, af_r, we_r, be_r, w1_r, ns_r, nd_r, t_r):
        ns_r[...] = lax.rsqrt(jnp.maximum(dego_r[...], 1.0))
        nd_r[...] = lax.rsqrt(jnp.maximum(degi_r[...], 1.0))
        wc = jnp.dot(we_r[...], w1_r[...], preferred_element_type=jnp.float32)
        bt = jnp.dot(be_r[...], w1_r[...], preferred_element_type=jnp.float32)
        t_r[...] = jnp.dot(af_r[...], wc,
                           preferred_element_type=jnp.float32) + bt

    return pl.pallas_call(
        body,
        out_shape=[
            jax.ShapeDtypeStruct((400, 128), jnp.float32),
            jax.ShapeDtypeStruct((400, 128), jnp.float32),
            jax.ShapeDtypeStruct((NT, D), jnp.float32),
        ],
    )(dego2, degi2, af_table, W_embed, b_embed2, W1)


def _sc_c_and_a(srcp, dstp, distp, at_h, ns_h, nd_h, zeros_h):
    mesh = plsc.VectorSubcoreMesh(core_axis_name="c", subcore_axis_name="s")
    EPW = EP // (NC * NS)

    @functools.partial(
        pl.kernel,
        out_type=[
            jax.ShapeDtypeStruct((NC * NP,), jnp.float32),
            jax.ShapeDtypeStruct((2 * 4 * ASIZE,), jnp.float32),
            jax.ShapeDtypeStruct((EP,), jnp.int32),
            jax.ShapeDtypeStruct((EP,), jnp.float32),
        ],
        mesh=mesh,
        scratch_types=[
            pltpu.VMEM((ZCH,), jnp.int32),
            pltpu.VMEM((ZCH,), jnp.int32),
            pltpu.VMEM((ZCH,), jnp.float32),
            pltpu.VMEM((ZCH,), jnp.float32),
            pltpu.VMEM((ZCH,), jnp.float32),
            pltpu.VMEM((ZCH,), jnp.int32),
            pltpu.VMEM((ZCH,), jnp.float32),
            pltpu.VMEM((ZCH,), jnp.int32),
            pltpu.VMEM((ZCH,), jnp.float32),
            pltpu.VMEM((ZCH,), jnp.int32),
            pltpu.VMEM((ZCH,), jnp.float32),
            pltpu.VMEM_SHARED((NP,), jnp.float32),
            pltpu.VMEM_SHARED((NP,), jnp.float32),
            pltpu.VMEM_SHARED((NP,), jnp.int32),
            pltpu.VMEM_SHARED((NP,), jnp.float32),
            pltpu.VMEM_SHARED((ASIZE,), jnp.float32),
        ],
        compiler_params=pltpu.CompilerParams(needs_layout_passes=False),
    )
    def sc2_kernel(src_r, dst_r, dist_r, at_r, ns_r, nd_r, zeros_r,
                   c_out, a_out, ei_out, ev_out,
                   sbuf, dbuf, rbuf, gns, gnd, gtyp, cvbuf,
                   pib0, pvb0, pib1, pvb1,
                   tabns, tabnd, tabi, c_sh, a_sh):
        c = lax.axis_index("c")
        s = lax.axis_index("s")
        w = c * NS + s
        stripe = pl.ds(s * ZCH, ZCH)

        pltpu.sync_copy(zeros_r.at[pl.ds(0, ZCH)], c_sh.at[stripe])
        pltpu.sync_copy(ns_r.at[stripe], tabns.at[stripe])
        pltpu.sync_copy(nd_r.at[stripe], tabnd.at[stripe])
        pltpu.sync_copy(at_r.at[stripe], tabi.at[stripe])
        plsc.subcore_barrier()
        for k in range(8):
            base = w * EPW + k * ZCH
            pltpu.sync_copy(src_r.at[pl.ds(base, ZCH)], sbuf)
            pltpu.sync_copy(dst_r.at[pl.ds(base, ZCH)], dbuf)
            pltpu.sync_copy(dist_r.at[pl.ds(base, ZCH)], rbuf)
            pltpu.sync_copy(tabns.at[sbuf], gns)
            pltpu.sync_copy(tabnd.at[dbuf], gnd)
            pltpu.sync_copy(tabi.at[sbuf], gtyp)

            def pbody(j, carry):
                sl = pl.ds(j * 16, 16)
                r16 = rbuf[sl]
                d16 = dbuf[sl]
                ew = jnp.exp(r16 * r16 * EW_SCALE)
                cvbuf[sl] = ew * gnd[sl]
                rbuf[sl] = ew * gns[sl]
                dbuf[sl] = d16 * NT + gtyp[sl]
                return carry

            lax.fori_loop(0, ZCH // 16, pbody, 0)
            pltpu.sync_copy(cvbuf, c_sh.at[sbuf], add=True)
            pltpu.sync_copy(dbuf, ei_out.at[pl.ds(base, ZCH)])
            pltpu.sync_copy(rbuf, ev_out.at[pl.ds(base, ZCH)])
        plsc.subcore_barrier()

        @pl.when(s == 0)
        def _():
            pltpu.sync_copy(c_sh, c_out.at[pl.ds(c * NP, NP)])

        for p in range(4):
            lo100 = p * ASIZE
            pltpu.sync_copy(zeros_r, a_sh.at[pl.ds(s * (ASIZE // NS),
                                                   ASIZE // NS)])
            plsc.subcore_barrier()
            bufs = [(pib0, pvb0), (pib1, pvb1)]
            for k in range(8):
                pib, pvb = bufs[k % 2]
                base = w * EPW + k * ZCH
                pltpu.sync_copy(ei_out.at[pl.ds(base, ZCH)], pib)
                pltpu.sync_copy(ev_out.at[pl.ds(base, ZCH)], pvb)

                def bbody(j, carry):
                    sl = pl.ds(j * 16, 16)
                    rel = pib[sl] - lo100
                    ok = (rel >= 0) & (rel < ASIZE)
                    pib[sl] = jnp.where(ok, rel, 0)
                    pvb[sl] = jnp.where(ok, pvb[sl], 0.0)
                    return carry

                lax.fori_loop(0, ZCH // 16, bbody, 0)
                pltpu.sync_copy(pvb, a_sh.at[pib], add=True)
            plsc.subcore_barrier()

            @pl.when(s == 0)
            def _():
                pltpu.sync_copy(a_sh,
                                a_out.at[pl.ds((c * 4 + p) * ASIZE, ASIZE)])

            plsc.subcore_barrier()

    return sc2_kernel(srcp, dstp, distp, at_h, ns_h, nd_h, zeros_h)


def _tc_finish(A, A1, T, nd1, ns1, c0, c1, b1_2, W2, b2_2):
    BN = 1024
    steps = NP // BN

    def body(a_r, a1_r, t_r, nd_r, ns_r, c0_r, c1_r, b1_r, w2_r, b2_r,
             out_r, acc):
        i = pl.program_id(0)

        @pl.when(i == 0)
        def _():
            acc[...] = jnp.zeros_like(acc)

        agg = jnp.dot(a_r[...] + a1_r[...], t_r[...],
                      preferred_element_type=jnp.float32)
        x = jnp.maximum(agg * nd_r[...] + b1_r[...], 0.0)
        cn = ns_r[...] * (c0_r[...] + c1_r[...])
        acc[...] += jnp.sum(cn * x, axis=0, keepdims=True)

        @pl.when(i == steps - 1)
        def _():
            out_r[...] = jnp.dot(acc[...] * (1.0 / N), w2_r[...],
                                 preferred_element_type=jnp.float32) + b2_r[...]

    return pl.pallas_call(
        body,
        grid=(steps,),
        in_specs=[
            pl.BlockSpec((BN, NT), lambda i: (i, 0)),
            pl.BlockSpec((BN, NT), lambda i: (i, 0)),
            pl.BlockSpec((NT, D), lambda i: (0, 0)),
            pl.BlockSpec((BN, 1), lambda i: (i, 0)),
            pl.BlockSpec((BN, 1), lambda i: (i, 0)),
            pl.BlockSpec((BN, 1), lambda i: (i, 0)),
            pl.BlockSpec((BN, 1), lambda i: (i, 0)),
            pl.BlockSpec((1, D), lambda i: (0, 0)),
            pl.BlockSpec((D, D), lambda i: (0, 0)),
            pl.BlockSpec((1, D), lambda i: (0, 0)),
        ],
        out_specs=pl.BlockSpec((1, D), lambda i: (0, 0)),
        out_shape=jax.ShapeDtypeStruct((1, D), jnp.float32),
        scratch_shapes=[pltpu.VMEM((1, D), jnp.float32)],
    )(A, A1, T, nd1, ns1, c0, c1, b1_2, W2, b2_2)


def kernel(atom_types, edge_index, distances, af_table, W_embed, b_embed,
           W1, b1, W2, b2):
    pad_e = EP - E
    srcp = jnp.concatenate([edge_index[0].astype(jnp.int32),
                            jnp.full((pad_e,), PAD_NODE, jnp.int32)])
    dstp = jnp.concatenate([edge_index[1].astype(jnp.int32),
                            jnp.full((pad_e,), PAD_NODE, jnp.int32)])
    distp = jnp.concatenate([distances.astype(jnp.float32),
                             jnp.full((pad_e,), 1e4, jnp.float32)])
    atp = jnp.concatenate([atom_types.astype(jnp.int32),
                           jnp.zeros((NP - N,), jnp.int32)])
    zeros_h = jnp.zeros((ZCH,), jnp.float32)
    zeros_big = jnp.zeros((ASIZE // NS,), jnp.float32)
    ones_h = jnp.ones((ZCH,), jnp.float32)

    dego, degi = _sc_degrees(srcp, dstp, ones_h, zeros_h)
    ns2, nd2, T = _tc_norms_table(dego.reshape(400, 128),
                                  degi.reshape(400, 128),
                                  af_table.astype(jnp.float32),
                                  W_embed.astype(jnp.float32),
                                  b_embed.reshape(1, D).astype(jnp.float32),
                                  W1.astype(jnp.float32))
    c2, A8, _, _ = _sc_c_and_a(srcp, dstp, distp,
                               atp, ns2.reshape(NP), nd2.reshape(NP),
                               zeros_big)
    A8r = A8.reshape(2, NP, NT)
    out = _tc_finish(A8r[0], A8r[1], T,
                     nd2.reshape(NP, 1), ns2.reshape(NP, 1),
                     c2[:NP].reshape(NP, 1), c2[NP:].reshape(NP, 1),
                     b1.reshape(1, D).astype(jnp.float32),
                     W2.astype(jnp.float32),
                     b2.reshape(1, D).astype(jnp.float32))
    return out


# speedup vs baseline: 1.0080x; 1.0080x over previous
import functools

import jax
import jax.numpy as jnp
from jax import lax
from jax.experimental import pallas as pl
from jax.experimental.pallas import tpu as pltpu
from jax.experimental.pallas import tpu_sc as plsc

N = 50000
E = 800000
NT = 100
D = 64
NP = 51200
EP = 819200
PAD_NODE = 50001
NC = 2
NS = 16
SHARD = 12800
ASIZE = SHARD * NT
ATRASH = ASIZE
ZCH = 3200
EW_SCALE = -1.0 / 64.0


def _sc_degrees(srcp, dstp, ones_h, zeros_h):
    mesh = plsc.VectorSubcoreMesh(core_axis_name="c", subcore_axis_name="s")

    @functools.partial(
        pl.kernel,
        out_type=[
            jax.ShapeDtypeStruct((NP,), jnp.float32),
            jax.ShapeDtypeStruct((NP,), jnp.float32),
        ],
        mesh=mesh,
        scratch_types=[
            pltpu.VMEM((ZCH,), jnp.int32),
            pltpu.VMEM((ZCH,), jnp.float32),
            pltpu.VMEM((ZCH,), jnp.float32),
            pltpu.VMEM_SHARED((NP,), jnp.float32),
        ],
        compiler_params=pltpu.CompilerParams(needs_layout_passes=False),
    )
    def deg_kernel(src_r, dst_r, ones_r, zeros_r, dego_r, degi_r,
                   ebuf, ones_v, zeros_v, hist):
        c = lax.axis_index("c")
        s = lax.axis_index("s")
        pltpu.sync_copy(ones_r, ones_v)
        pltpu.sync_copy(zeros_r, zeros_v)
        pltpu.sync_copy(zeros_v, hist.at[pl.ds(s * ZCH, ZCH)])
        plsc.subcore_barrier()
        base = s * (EP // NS)
        for k in range(16):
            @pl.when(c == 0)
            def _():
                pltpu.sync_copy(src_r.at[pl.ds(base + k * ZCH, ZCH)], ebuf)

            @pl.when(c == 1)
            def _():
                pltpu.sync_copy(dst_r.at[pl.ds(base + k * ZCH, ZCH)], ebuf)

            pltpu.sync_copy(ones_v, hist.at[ebuf], add=True)
        plsc.subcore_barrier()

        @pl.when((c == 0) & (s == 0))
        def _():
            pltpu.sync_copy(hist, dego_r)

        @pl.when((c == 1) & (s == 0))
        def _():
            pltpu.sync_copy(hist, degi_r)

    return deg_kernel(srcp, dstp, ones_h, zeros_h)


def _tc_norms_table(dego2, degi2, af_table, W_embed, b_embed2, W1):
    def body(dego_r, degi_r, af_r, we_r, be_r, w1_r, ns_r, nd_r, t_r):
        ns_r[...] = lax.rsqrt(jnp.maximum(dego_r[...], 1.0))
        nd_r[...] = lax.rsqrt(jnp.maximum(degi_r[...], 1.0))
        wc = jnp.dot(we_r[...], w1_r[...], preferred_element_type=jnp.float32)
        bt = jnp.dot(be_r[...], w1_r[...], preferred_element_type=jnp.float32)
        t_r[...] = jnp.dot(af_r[...], wc,
                           preferred_element_type=jnp.float32) + bt

    return pl.pallas_call(
        body,
        out_shape=[
            jax.ShapeDtypeStruct((400, 128), jnp.float32),
            jax.ShapeDtypeStruct((400, 128), jnp.float32),
            jax.ShapeDtypeStruct((NT, D), jnp.float32),
        ],
    )(dego2, degi2, af_table, W_embed, b_embed2, W1)


def _sc_c_and_a(srcp, dstp, distp, at_h, ns_h, nd_h, zeros_h):
    mesh = plsc.VectorSubcoreMesh(core_axis_name="c", subcore_axis_name="s")
    EPW = EP // (NC * NS)

    @functools.partial(
        pl.kernel,
        out_type=[
            jax.ShapeDtypeStruct((NC * NP,), jnp.float32),
            jax.ShapeDtypeStruct((2 * 4 * ASIZE,), jnp.float32),
            jax.ShapeDtypeStruct((EP,), jnp.int32),
            jax.ShapeDtypeStruct((EP,), jnp.float32),
        ],
        mesh=mesh,
        scratch_types=[
            pltpu.VMEM((ZCH,), jnp.int32),
            pltpu.VMEM((ZCH,), jnp.int32),
            pltpu.VMEM((ZCH,), jnp.float32),
            pltpu.VMEM((ZCH,), jnp.float32),
            pltpu.VMEM((ZCH,), jnp.float32),
            pltpu.VMEM((ZCH,), jnp.int32),
            pltpu.VMEM((ZCH,), jnp.float32),
            pltpu.VMEM((ZCH,), jnp.int32),
            pltpu.VMEM((ZCH,), jnp.float32),
            pltpu.VMEM((ZCH,), jnp.int32),
            pltpu.VMEM((ZCH,), jnp.float32),
            pltpu.VMEM_SHARED((NP,), jnp.float32),
            pltpu.VMEM_SHARED((NP,), jnp.float32),
            pltpu.VMEM_SHARED((NP,), jnp.int32),
            pltpu.VMEM_SHARED((NP,), jnp.float32),
            pltpu.VMEM_SHARED((ASIZE,), jnp.float32),
        ],
        compiler_params=pltpu.CompilerParams(needs_layout_passes=False),
    )
    def sc2_kernel(src_r, dst_r, dist_r, at_r, ns_r, nd_r, zeros_r,
                   c_out, a_out, ei_out, ev_out,
                   sbuf, dbuf, rbuf, gns, gnd, gtyp, cvbuf,
                   pib0, pvb0, pib1, pvb1,
                   tabns, tabnd, tabi, c_sh, a_sh):
        c = lax.axis_index("c")
        s = lax.axis_index("s")
        w = c * NS + s
        stripe = pl.ds(s * ZCH, ZCH)

        pltpu.sync_copy(zeros_r.at[pl.ds(0, ZCH)], c_sh.at[stripe])
        pltpu.sync_copy(ns_r.at[stripe], tabns.at[stripe])
        pltpu.sync_copy(nd_r.at[stripe], tabnd.at[stripe])
        pltpu.sync_copy(at_r.at[stripe], tabi.at[stripe])
        plsc.subcore_barrier()
        for k in range(8):
            base = w * EPW + k * ZCH
            pltpu.sync_copy(src_r.at[pl.ds(base, ZCH)], sbuf)
            pltpu.sync_copy(dst_r.at[pl.ds(base, ZCH)], dbuf)
            pltpu.sync_copy(dist_r.at[pl.ds(base, ZCH)], rbuf)
            pltpu.sync_copy(tabns.at[sbuf], gns)
            pltpu.sync_copy(tabnd.at[dbuf], gnd)
            pltpu.sync_copy(tabi.at[sbuf], gtyp)

            @plsc.parallel_loop(0, ZCH, step=16)
            def pbody(i):
                sl = pl.ds(i, 16)
                r16 = rbuf[sl]
                d16 = dbuf[sl]
                ew = jnp.exp(r16 * r16 * EW_SCALE)
                cvbuf[sl] = ew * gnd[sl]
                rbuf[sl] = ew * gns[sl]
                dbuf[sl] = d16 * NT + gtyp[sl]
            pltpu.sync_copy(cvbuf, c_sh.at[sbuf], add=True)
            pltpu.sync_copy(dbuf, ei_out.at[pl.ds(base, ZCH)])
            pltpu.sync_copy(rbuf, ev_out.at[pl.ds(base, ZCH)])
        plsc.subcore_barrier()

        @pl.when(s == 0)
        def _():
            pltpu.sync_copy(c_sh, c_out.at[pl.ds(c * NP, NP)])

        for p in range(4):
            lo100 = p * ASIZE
            pltpu.sync_copy(zeros_r, a_sh.at[pl.ds(s * (ASIZE // NS),
                                                   ASIZE // NS)])
            plsc.subcore_barrier()
            bufs = [(pib0, pvb0), (pib1, pvb1)]
            for k in range(8):
                pib, pvb = bufs[k % 2]
                base = w * EPW + k * ZCH
                pltpu.sync_copy(ei_out.at[pl.ds(base, ZCH)], pib)
                pltpu.sync_copy(ev_out.at[pl.ds(base, ZCH)], pvb)

                @plsc.parallel_loop(0, ZCH, step=16)
                def bbody(i):
                    sl = pl.ds(i, 16)
                    rel = pib[sl] - lo100
                    ok = (rel >= 0) & (rel < ASIZE)
                    pib[sl] = jnp.where(ok, rel, 0)
                    pvb[sl] = jnp.where(ok, pvb[sl], 0.0)
                pltpu.sync_copy(pvb, a_sh.at[pib], add=True)
            plsc.subcore_barrier()

            @pl.when(s == 0)
            def _():
                pltpu.sync_copy(a_sh,
                                a_out.at[pl.ds((c * 4 + p) * ASIZE, ASIZE)])

            plsc.subcore_barrier()

    return sc2_kernel(srcp, dstp, distp, at_h, ns_h, nd_h, zeros_h)


def _tc_finish(A, A1, T, nd1, ns1, c0, c1, b1_2, W2, b2_2):
    BN = 1024
    steps = NP // BN

    def body(a_r, a1_r, t_r, nd_r, ns_r, c0_r, c1_r, b1_r, w2_r, b2_r,
             out_r, acc):
        i = pl.program_id(0)

        @pl.when(i == 0)
        def _():
            acc[...] = jnp.zeros_like(acc)

        agg = jnp.dot(a_r[...] + a1_r[...], t_r[...],
                      preferred_element_type=jnp.float32)
        x = jnp.maximum(agg * nd_r[...] + b1_r[...], 0.0)
        cn = ns_r[...] * (c0_r[...] + c1_r[...])
        acc[...] += jnp.sum(cn * x, axis=0, keepdims=True)

        @pl.when(i == steps - 1)
        def _():
            out_r[...] = jnp.dot(acc[...] * (1.0 / N), w2_r[...],
                                 preferred_element_type=jnp.float32) + b2_r[...]

    return pl.pallas_call(
        body,
        grid=(steps,),
        in_specs=[
            pl.BlockSpec((BN, NT), lambda i: (i, 0)),
            pl.BlockSpec((BN, NT), lambda i: (i, 0)),
            pl.BlockSpec((NT, D), lambda i: (0, 0)),
            pl.BlockSpec((BN, 1), lambda i: (i, 0)),
            pl.BlockSpec((BN, 1), lambda i: (i, 0)),
            pl.BlockSpec((BN, 1), lambda i: (i, 0)),
            pl.BlockSpec((BN, 1), lambda i: (i, 0)),
            pl.BlockSpec((1, D), lambda i: (0, 0)),
            pl.BlockSpec((D, D), lambda i: (0, 0)),
            pl.BlockSpec((1, D), lambda i: (0, 0)),
        ],
        out_specs=pl.BlockSpec((1, D), lambda i: (0, 0)),
        out_shape=jax.ShapeDtypeStruct((1, D), jnp.float32),
        scratch_shapes=[pltpu.VMEM((1, D), jnp.float32)],
    )(A, A1, T, nd1, ns1, c0, c1, b1_2, W2, b2_2)


def kernel(atom_types, edge_index, distances, af_table, W_embed, b_embed,
           W1, b1, W2, b2):
    pad_e = EP - E
    srcp = jnp.concatenate([edge_index[0].astype(jnp.int32),
                            jnp.full((pad_e,), PAD_NODE, jnp.int32)])
    dstp = jnp.concatenate([edge_index[1].astype(jnp.int32),
                            jnp.full((pad_e,), PAD_NODE, jnp.int32)])
    distp = jnp.concatenate([distances.astype(jnp.float32),
                             jnp.full((pad_e,), 1e4, jnp.float32)])
    atp = jnp.concatenate([atom_types.astype(jnp.int32),
                           jnp.zeros((NP - N,), jnp.int32)])
    zeros_h = jnp.zeros((ZCH,), jnp.float32)
    zeros_big = jnp.zeros((ASIZE // NS,), jnp.float32)
    ones_h = jnp.ones((ZCH,), jnp.float32)

    dego, degi = _sc_degrees(srcp, dstp, ones_h, zeros_h)
    ns2, nd2, T = _tc_norms_table(dego.reshape(400, 128),
                                  degi.reshape(400, 128),
                                  af_table.astype(jnp.float32),
                                  W_embed.astype(jnp.float32),
                                  b_embed.reshape(1, D).astype(jnp.float32),
                                  W1.astype(jnp.float32))
    c2, A8, _, _ = _sc_c_and_a(srcp, dstp, distp,
                               atp, ns2.reshape(NP), nd2.reshape(NP),
                               zeros_big)
    A8r = A8.reshape(2, NP, NT)
    out = _tc_finish(A8r[0], A8r[1], T,
                     nd2.reshape(NP, 1), ns2.reshape(NP, 1),
                     c2[:NP].reshape(NP, 1), c2[NP:].reshape(NP, 1),
                     b1.reshape(1, D).astype(jnp.float32),
                     W2.astype(jnp.float32),
                     b2.reshape(1, D).astype(jnp.float32))
    return out


# speedup vs baseline: 1.1197x; 1.1108x over previous
import functools

import jax
import jax.numpy as jnp
from jax import lax
from jax.experimental import pallas as pl
from jax.experimental.pallas import tpu as pltpu
from jax.experimental.pallas import tpu_sc as plsc

N = 50000
E = 800000
NT = 100
D = 64
NP = 51200
EP = 819200
PAD_NODE = 50001
NC = 2
NS = 16
SHARD = 12800
ASIZE = SHARD * NT
ATRASH = ASIZE
ZCH = 3200
EW_SCALE = -1.0 / 64.0


def _sc_degrees(srcp, dstp, ones_h, zeros_h):
    mesh = plsc.VectorSubcoreMesh(core_axis_name="c", subcore_axis_name="s")

    @functools.partial(
        pl.kernel,
        out_type=[
            jax.ShapeDtypeStruct((NP,), jnp.float32),
            jax.ShapeDtypeStruct((NP,), jnp.float32),
        ],
        mesh=mesh,
        scratch_types=[
            pltpu.VMEM((ZCH,), jnp.int32),
            pltpu.VMEM((ZCH,), jnp.float32),
            pltpu.VMEM((ZCH,), jnp.float32),
            pltpu.VMEM_SHARED((NP,), jnp.float32),
        ],
        compiler_params=pltpu.CompilerParams(needs_layout_passes=False),
    )
    def deg_kernel(src_r, dst_r, ones_r, zeros_r, dego_r, degi_r,
                   ebuf, ones_v, zeros_v, hist):
        c = lax.axis_index("c")
        s = lax.axis_index("s")
        pltpu.sync_copy(ones_r, ones_v)
        pltpu.sync_copy(zeros_r, zeros_v)
        pltpu.sync_copy(zeros_v, hist.at[pl.ds(s * ZCH, ZCH)])
        plsc.subcore_barrier()
        base = s * (EP // NS)
        for k in range(16):
            @pl.when(c == 0)
            def _():
                pltpu.sync_copy(src_r.at[pl.ds(base + k * ZCH, ZCH)], ebuf)

            @pl.when(c == 1)
            def _():
                pltpu.sync_copy(dst_r.at[pl.ds(base + k * ZCH, ZCH)], ebuf)

            pltpu.sync_copy(ones_v, hist.at[ebuf], add=True)
        plsc.subcore_barrier()

        @pl.when((c == 0) & (s == 0))
        def _():
            pltpu.sync_copy(hist, dego_r)

        @pl.when((c == 1) & (s == 0))
        def _():
            pltpu.sync_copy(hist, degi_r)

    return deg_kernel(srcp, dstp, ones_h, zeros_h)


def _tc_norms_table(dego2, degi2, af_table, W_embed, b_embed2, W1):
    def body(dego_r, degi_r, af_r, we_r, be_r, w1_r, ns_r, nd_r, t_r):
        ns_r[...] = lax.rsqrt(jnp.maximum(dego_r[...], 1.0))
        nd_r[...] = lax.rsqrt(jnp.maximum(degi_r[...], 1.0))
        wc = jnp.dot(we_r[...], w1_r[...], preferred_element_type=jnp.float32)
        bt = jnp.dot(be_r[...], w1_r[...], preferred_element_type=jnp.float32)
        t_r[...] = jnp.dot(af_r[...], wc,
                           preferred_element_type=jnp.float32) + bt

    return pl.pallas_call(
        body,
        out_shape=[
            jax.ShapeDtypeStruct((400, 128), jnp.float32),
            jax.ShapeDtypeStruct((400, 128), jnp.float32),
            jax.ShapeDtypeStruct((NT, D), jnp.float32),
        ],
    )(dego2, degi2, af_table, W_embed, b_embed2, W1)


def _sc_c_and_a(srcp, dstp, distp, at_h, ns_h, nd_h, zeros_h):
    mesh = plsc.VectorSubcoreMesh(core_axis_name="c", subcore_axis_name="s")
    EPW = EP // (NC * NS)

    @functools.partial(
        pl.kernel,
        out_type=[
            jax.ShapeDtypeStruct((NC * NP,), jnp.float32),
            jax.ShapeDtypeStruct((4 * ASIZE,), jnp.float32),
        ],
        mesh=mesh,
        scratch_types=[
            pltpu.VMEM((ZCH,), jnp.int32),
            pltpu.VMEM((ZCH,), jnp.int32),
            pltpu.VMEM((ZCH,), jnp.float32),
            pltpu.VMEM((ZCH,), jnp.float32),
            pltpu.VMEM((ZCH,), jnp.int32),
            pltpu.VMEM((ZCH,), jnp.int32),
            pltpu.VMEM((ZCH,), jnp.float32),
            pltpu.VMEM((ZCH,), jnp.float32),
            pltpu.VMEM_SHARED((NP,), jnp.float32),
            pltpu.VMEM_SHARED((NP,), jnp.int32),
            pltpu.VMEM_SHARED((NP,), jnp.float32),
            pltpu.VMEM_SHARED((ASIZE + 128,), jnp.float32),
        ],
        compiler_params=pltpu.CompilerParams(needs_layout_passes=False),
    )
    def sc2_kernel(src_r, dst_r, dist_r, at_r, ns_r, nd_r, zeros_r,
                   c_out, a_out,
                   sbuf, dbuf, rbuf, gval, gtyp, ibuf, vbuf, zeros_v,
                   tabf, tabi, c_sh, a_sh):
        c = lax.axis_index("c")
        s = lax.axis_index("s")
        w = c * NS + s
        stripe = pl.ds(s * ZCH, ZCH)
        pltpu.sync_copy(zeros_r, zeros_v)

        def load_chunk(base):
            pltpu.sync_copy(src_r.at[pl.ds(base, ZCH)], sbuf)
            pltpu.sync_copy(dst_r.at[pl.ds(base, ZCH)], dbuf)
            pltpu.sync_copy(dist_r.at[pl.ds(base, ZCH)], rbuf)

        pltpu.sync_copy(zeros_v, c_sh.at[stripe])
        pltpu.sync_copy(nd_r.at[stripe], tabf.at[stripe])
        plsc.subcore_barrier()
        for k in range(8):
            load_chunk(w * EPW + k * ZCH)
            pltpu.sync_copy(tabf.at[dbuf], gval)

            def cbody(j, carry):
                sl = pl.ds(j * 16, 16)
                r16 = rbuf[sl]
                ew = jnp.exp(r16 * r16 * EW_SCALE)
                vbuf[sl] = ew * gval[sl]
                return carry

            lax.fori_loop(0, ZCH // 16, cbody, 0)
            pltpu.sync_copy(vbuf, c_sh.at[sbuf], add=True)
        plsc.subcore_barrier()

        @pl.when(s == 0)
        def _():
            pltpu.sync_copy(c_sh, c_out.at[pl.ds(c * NP, NP)])

        pltpu.sync_copy(ns_r.at[stripe], tabf.at[stripe])
        pltpu.sync_copy(at_r.at[stripe], tabi.at[stripe])
        for p in range(2):
            lo = (2 * p + c) * SHARD
            for i in range(25):
                pltpu.sync_copy(zeros_v,
                                a_sh.at[pl.ds((s + i * NS) * ZCH, ZCH)])
            plsc.subcore_barrier()
            for k in range(16):
                load_chunk(s * (EP // NS) + k * ZCH)
                pltpu.sync_copy(tabf.at[sbuf], gval)
                pltpu.sync_copy(tabi.at[sbuf], gtyp)

                def abody(j, carry):
                    sl = pl.ds(j * 16, 16)
                    d16 = dbuf[sl]
                    r16 = rbuf[sl]
                    ew = jnp.exp(r16 * r16 * EW_SCALE)
                    rel = d16 - lo
                    ok = (rel >= 0) & (rel < SHARD)
                    idx = jnp.where(ok, rel * NT + gtyp[sl], ATRASH)
                    vbuf[sl] = gval[sl] * ew
                    ibuf[sl] = idx
                    return carry

                lax.fori_loop(0, ZCH // 16, abody, 0)
                pltpu.sync_copy(vbuf, a_sh.at[ibuf], add=True)
            plsc.subcore_barrier()

            @pl.when(s == 0)
            def _():
                pltpu.sync_copy(a_sh.at[pl.ds(0, ASIZE)],
                                a_out.at[pl.ds((2 * p + c) * ASIZE, ASIZE)])

            plsc.subcore_barrier()

    return sc2_kernel(srcp, dstp, distp, at_h, ns_h, nd_h, zeros_h)


def _tc_finish(A, T, nd1, ns1, c0, c1, b1_2, W2, b2_2):
    BN = 1024
    steps = NP // BN

    def body(a_r, t_r, nd_r, ns_r, c0_r, c1_r, b1_r, w2_r, b2_r,
             out_r, acc):
        i = pl.program_id(0)

        @pl.when(i == 0)
        def _():
            acc[...] = jnp.zeros_like(acc)

        agg = jnp.dot(a_r[...], t_r[...], preferred_element_type=jnp.float32)
        x = jnp.maximum(agg * nd_r[...] + b1_r[...], 0.0)
        cn = ns_r[...] * (c0_r[...] + c1_r[...])
        acc[...] += jnp.sum(cn * x, axis=0, keepdims=True)

        @pl.when(i == steps - 1)
        def _():
            out_r[...] = jnp.dot(acc[...] * (1.0 / N), w2_r[...],
                                 preferred_element_type=jnp.float32) + b2_r[...]

    return pl.pallas_call(
        body,
        grid=(steps,),
        in_specs=[
            pl.BlockSpec((BN, NT), lambda i: (i, 0)),
            pl.BlockSpec((NT, D), lambda i: (0, 0)),
            pl.BlockSpec((BN, 1), lambda i: (i, 0)),
            pl.BlockSpec((BN, 1), lambda i: (i, 0)),
            pl.BlockSpec((BN, 1), lambda i: (i, 0)),
            pl.BlockSpec((BN, 1), lambda i: (i, 0)),
            pl.BlockSpec((1, D), lambda i: (0, 0)),
            pl.BlockSpec((D, D), lambda i: (0, 0)),
            pl.BlockSpec((1, D), lambda i: (0, 0)),
        ],
        out_specs=pl.BlockSpec((1, D), lambda i: (0, 0)),
        out_shape=jax.ShapeDtypeStruct((1, D), jnp.float32),
        scratch_shapes=[pltpu.VMEM((1, D), jnp.float32)],
    )(A, T, nd1, ns1, c0, c1, b1_2, W2, b2_2)


def kernel(atom_types, edge_index, distances, af_table, W_embed, b_embed,
           W1, b1, W2, b2):
    pad_e = EP - E
    srcp = jnp.concatenate([edge_index[0].astype(jnp.int32),
                            jnp.full((pad_e,), PAD_NODE, jnp.int32)])
    dstp = jnp.concatenate([edge_index[1].astype(jnp.int32),
                            jnp.full((pad_e,), PAD_NODE, jnp.int32)])
    distp = jnp.concatenate([distances.astype(jnp.float32),
                             jnp.full((pad_e,), 1e4, jnp.float32)])
    atp = jnp.concatenate([atom_types.astype(jnp.int32),
                           jnp.zeros((NP - N,), jnp.int32)])
    zeros_h = jnp.zeros((ZCH,), jnp.float32)
    ones_h = jnp.ones((ZCH,), jnp.float32)

    dego, degi = _sc_degrees(srcp, dstp, ones_h, zeros_h)
    ns2, nd2, T = _tc_norms_table(dego.reshape(400, 128),
                                  degi.reshape(400, 128),
                                  af_table.astype(jnp.float32),
                                  W_embed.astype(jnp.float32),
                                  b_embed.reshape(1, D).astype(jnp.float32),
                                  W1.astype(jnp.float32))
    c2, A4 = _sc_c_and_a(srcp, dstp, distp,
                         atp, ns2.reshape(NP), nd2.reshape(NP), zeros_h)
    out = _tc_finish(A4.reshape(NP, NT), T,
                     nd2.reshape(NP, 1), ns2.reshape(NP, 1),
                     c2[:NP].reshape(NP, 1), c2[NP:].reshape(NP, 1),
                     b1.reshape(1, D).astype(jnp.float32),
                     W2.astype(jnp.float32),
                     b2.reshape(1, D).astype(jnp.float32))
    return out
